# baseline (speedup 1.0000x reference)
import jax
import jax.numpy as jnp
from jax import lax
from jax.experimental import pallas as pl
from jax.experimental.pallas import tpu as pltpu
from jax.experimental.pallas import tpu_sc as plsc

_Y_CHUNK = 4096
_NW = 32
_LANES = 16


def _split_bf16(a):
    hi = a.astype(jnp.bfloat16)
    lo = (a - hi.astype(jnp.float32)).astype(jnp.bfloat16)
    return hi, lo


def _dotn(a, b):
    return jax.lax.dot_general(a, b, (((1,), (1,)), ((), ())),
                               preferred_element_type=jnp.float32)


def _dn_argmax_step(x_ref, w_ref, age_ref, idx_ref, max_ref, gidx_ref):
    i = pl.program_id(0)
    nsteps = pl.num_programs(0)

    @pl.when(i == 0)
    def _init():
        max_ref[...] = jnp.full_like(max_ref, -jnp.inf)
        gidx_ref[...] = jnp.zeros_like(gidx_ref)

    xs = x_ref[...]
    w = w_ref[...]
    age = age_ref[...]
    chunk = w.shape[0]
    b = xs.shape[0] // 2

    wh, wl = _split_bf16(w)
    p1 = _dotn(xs, wh)
    p2 = _dotn(xs, wl)
    dots = (p1[:b] + p1[b:]) + (p2[:b] + p2[b:])

    ones_x = jnp.ones((1, w.shape[1]), jnp.bfloat16)
    sqh, sql = _split_bf16(w * w)
    wssq = _dotn(ones_x, sqh) + _dotn(ones_x, sql)

    recip = 1.0 / jnp.maximum(jnp.sqrt(wssq), 1e-12)
    act = jnp.where(age >= 1.0, 1.0, 0.0)
    y_pre = dots * (recip * act)

    local_max = jnp.max(y_pre, axis=1, keepdims=True)
    iota = lax.broadcasted_iota(jnp.int32, y_pre.shape, 1)
    eq = y_pre == local_max
    first = jnp.min(jnp.where(eq, iota, chunk), axis=1, keepdims=True)

    better = local_max > max_ref[...]
    max_ref[...] = jnp.where(better, local_max, max_ref[...])
    gidx_ref[...] = jnp.where(better, i * chunk + first, gidx_ref[...])

    @pl.when(i == nsteps - 1)
    def _fin():
        idx_ref[...] = gidx_ref[...]


def _allsum(v):
    lanes = jnp.arange(_LANES, dtype=jnp.int32)
    dnums = lax.GatherDimensionNumbers(offset_dims=(), collapsed_slice_dims=(0,),
                                       start_index_map=(0,))
    for sh in (8, 4, 2, 1):
        v = v + lax.gather(v, (lanes ^ sh)[:, None], dnums, slice_sizes=(1,),
                           mode=lax.GatherScatterMode.PROMISE_IN_BOUNDS)
    return v


def _sc_ssq(y2z_hbm, part_hbm, tile, acc_v):
    c = lax.axis_index("c")
    s = lax.axis_index("s")
    wid = s * 2 + c
    z_num, y_num = y2z_hbm.shape
    stripe = y_num // _NW
    base = wid * stripe
    for z in range(z_num):
        pltpu.sync_copy(y2z_hbm.at[z, pl.ds(base, stripe)], tile.at[z])
    lanes = jnp.arange(_LANES, dtype=jnp.int32)
    part = jnp.zeros((_LANES,), jnp.float32)
    for z in range(z_num):
        acc = jnp.zeros((_LANES,), jnp.float32)
        for k in range(stripe // _LANES):
            v = tile[z, pl.ds(k * _LANES, _LANES)]
            acc = acc + v * v
        part = jnp.where(lanes == z, _allsum(acc), part)
    acc_v[...] = part
    pltpu.sync_copy(acc_v, part_hbm.at[wid])


def _make_sc_gather(y_num, z_num):
    def _sc_gather(y2z_flat_hbm, idx_hbm, part_hbm, out_hbm, idx_v, part_v,
                   g_v, res_v, sem):
        c = lax.axis_index("c")
        s = lax.axis_index("s")
        b = s * 2 + c
        pltpu.sync_copy(idx_hbm, idx_v)
        pltpu.sync_copy(part_hbm, part_v)
        ssq = jnp.zeros((_LANES,), jnp.float32)
        for j in range(_NW):
            ssq = ssq + part_v[j]
        ssq = jnp.maximum(ssq, 1e-24)
        bits = lax.bitcast_convert_type(ssq, jnp.int32)
        r = lax.bitcast_convert_type(jnp.int32(0x5F3759DF) - (bits >> 1),
                                     jnp.float32)
        for _ in range(4):
            r = r * (1.5 - 0.5 * ssq * r * r)

        lanes = jnp.arange(_LANES, dtype=jnp.int32)
        lo = idx_v[pl.ds(0, _LANES)]
        hi = idx_v[pl.ds(_LANES, _LANES)]
        sel = jnp.where(b < _LANES, lo, hi)
        idx_b = _allsum(jnp.where(lanes == b % _LANES, sel, 0))
        zcl = jnp.minimum(lanes, jnp.int32(z_num - 1))
        flat = zcl * y_num + idx_b
        pltpu.async_copy(y2z_flat_hbm.at[flat], g_v, sem).wait()
        res_v[...] = g_v[...] * r
        pltpu.sync_copy(res_v, out_hbm.at[b])

    return _sc_gather


def kernel(x, z, per_item, x2y_w, z2y_w, y2z_w, y_neuron_age):
    batch = x.shape[0]
    xf = x.reshape(batch, -1)
    x_dim = xf.shape[1]
    y_num = x2y_w.shape[0]
    z_num = y2z_w.shape[0]
    grid = y_num // _Y_CHUNK

    xh = xf.astype(jnp.bfloat16)
    xl = (xf - xh.astype(jnp.float32)).astype(jnp.bfloat16)
    xs = jnp.concatenate([xh, xl], axis=0)

    idx = pl.pallas_call(
        _dn_argmax_step,
        grid=(grid,),
        in_specs=[
            pl.BlockSpec((2 * batch, x_dim), lambda i: (0, 0)),
            pl.BlockSpec((_Y_CHUNK, x_dim), lambda i: (i, 0)),
            pl.BlockSpec((1, _Y_CHUNK), lambda i: (0, i)),
        ],
        out_specs=pl.BlockSpec((batch, 1), lambda i: (0, 0)),
        out_shape=jax.ShapeDtypeStruct((batch, 1), jnp.int32),
        scratch_shapes=[
            pltpu.VMEM((batch, 1), jnp.float32),
            pltpu.VMEM((batch, 1), jnp.int32),
        ],
    )(xs, x2y_w, y_neuron_age)

    mesh = plsc.VectorSubcoreMesh(core_axis_name="c", subcore_axis_name="s")

    partials = pl.kernel(
        _sc_ssq,
        mesh=mesh,
        out_type=jax.ShapeDtypeStruct((_NW, _LANES), jnp.float32),
        scratch_types=[
            pltpu.VMEM((z_num, y_num // _NW), jnp.float32),
            pltpu.VMEM((_LANES,), jnp.float32),
        ],
    )(y2z_w)

    y2z_flat = y2z_w.reshape(-1)

    out16 = pl.kernel(
        _make_sc_gather(y_num, z_num),
        mesh=mesh,
        out_type=jax.ShapeDtypeStruct((batch, _LANES), jnp.float32),
        scratch_types=[
            pltpu.VMEM((batch,), jnp.int32),
            pltpu.VMEM((_NW, _LANES), jnp.float32),
            pltpu.VMEM((_LANES,), jnp.float32),
            pltpu.VMEM((_LANES,), jnp.float32),
            pltpu.SemaphoreType.DMA,
        ],
    )(y2z_flat, idx.reshape(batch), partials)

    return out16[:, :z_num]


# speedup vs baseline: 1.5863x; 1.5863x over previous
import jax
import jax.numpy as jnp
from jax.experimental import pallas as pl
from jax.experimental.pallas import tpu as pltpu

_Y_CHUNK = 4096


def _split_bf16(a):
    hi = a.astype(jnp.bfloat16)
    lo = (a - hi.astype(jnp.float32)).astype(jnp.bfloat16)
    return hi, lo


def _dotn(a, b):
    return jax.lax.dot_general(a, b, (((1,), (1,)), ((), ())),
                               preferred_element_type=jnp.float32)


def _dn_step(x_ref, w_ref, age_ref, y2z_ref, out_ref, max_ref, cand_ref, ssq_ref):
    i = pl.program_id(0)
    nsteps = pl.num_programs(0)

    @pl.when(i == 0)
    def _init():
        max_ref[...] = jnp.full_like(max_ref, -jnp.inf)
        cand_ref[...] = jnp.zeros_like(cand_ref)
        ssq_ref[...] = jnp.zeros_like(ssq_ref)

    xs = x_ref[...]
    b = xs.shape[0] // 2
    chunk = w_ref.shape[0]
    half = chunk // 2

    for h in range(2):
        w = w_ref[h * half:(h + 1) * half, :]
        y2z = y2z_ref[:, h * half:(h + 1) * half]
        age = age_ref[:, h * half:(h + 1) * half]

        wh, wl = _split_bf16(w)
        p1 = _dotn(xs, wh)
        p2 = _dotn(xs, wl)
        dots = (p1[:b] + p1[b:]) + (p2[:b] + p2[b:])

        ones_x = jnp.ones((1, w.shape[1]), jnp.bfloat16)
        sqh, sql = _split_bf16(w * w)
        wssq = _dotn(ones_x, sqh) + _dotn(ones_x, sql)

        recip = 1.0 / jnp.maximum(jnp.sqrt(wssq), 1e-12)
        act = jnp.where(age >= 1.0, 1.0, 0.0)
        y_pre = dots * (recip * act)

        local_max = jnp.max(y_pre, axis=1, keepdims=True)
        iota = jax.lax.broadcasted_iota(jnp.int32, y_pre.shape, 1)
        eq = y_pre == local_max
        first = jnp.min(jnp.where(eq, iota, half), axis=1, keepdims=True)
        onehot = (iota == first).astype(jnp.float32)

        cand = _dotn(onehot, y2z)
        better = local_max > max_ref[...]
        max_ref[...] = jnp.where(better, local_max, max_ref[...])
        cand_ref[...] = jnp.where(better, cand, cand_ref[...])

        ones_y = jnp.ones((1, half), jnp.float32)
        ssq_ref[...] += _dotn(ones_y, y2z * y2z)

    @pl.when(i == nsteps - 1)
    def _fin():
        zn = jnp.maximum(jnp.sqrt(ssq_ref[...]), 1e-12)
        out_ref[...] = cand_ref[...] / zn


def kernel(x, z, per_item, x2y_w, z2y_w, y2z_w, y_neuron_age):
    batch = x.shape[0]
    xf = x.reshape(batch, -1)
    x_dim = xf.shape[1]
    y_num = x2y_w.shape[0]
    z_num = y2z_w.shape[0]
    grid = y_num // _Y_CHUNK

    xh = xf.astype(jnp.bfloat16)
    xl = (xf - xh.astype(jnp.float32)).astype(jnp.bfloat16)
    xs = jnp.concatenate([xh, xl], axis=0)

    return pl.pallas_call(
        _dn_step,
        grid=(grid,),
        in_specs=[
            pl.BlockSpec((2 * batch, x_dim), lambda i: (0, 0)),
            pl.BlockSpec((_Y_CHUNK, x_dim), lambda i: (i, 0)),
            pl.BlockSpec((1, _Y_CHUNK), lambda i: (0, i)),
            pl.BlockSpec((z_num, _Y_CHUNK), lambda i: (0, i)),
        ],
        out_specs=pl.BlockSpec((batch, z_num), lambda i: (0, 0)),
        out_shape=jax.ShapeDtypeStruct((batch, z_num), jnp.float32),
        scratch_shapes=[
            pltpu.VMEM((batch, 1), jnp.float32),
            pltpu.VMEM((batch, z_num), jnp.float32),
            pltpu.VMEM((1, z_num), jnp.float32),
        ],
    )(xs, x2y_w, y_neuron_age, y2z_w)


# speedup vs baseline: 1.7092x; 1.0775x over previous
import jax
import jax.numpy as jnp
from jax.experimental import pallas as pl
from jax.experimental.pallas import tpu as pltpu

_Y_CHUNK = 4096


def _split_bf16(a):
    hi = a.astype(jnp.bfloat16)
    lo = (a - hi.astype(jnp.float32)).astype(jnp.bfloat16)
    return hi, lo


def _dotn(a, b):
    return jax.lax.dot_general(a, b, (((1,), (1,)), ((), ())),
                               preferred_element_type=jnp.float32)


def _reduce_chunk(y_pre, y2z, max_ref, cand_ref, ssq_ref):
    chunk = y_pre.shape[1]
    local_max = jnp.max(y_pre, axis=1, keepdims=True)
    iota = jax.lax.broadcasted_iota(jnp.int32, y_pre.shape, 1)
    eq = y_pre == local_max
    first = jnp.min(jnp.where(eq, iota, chunk), axis=1, keepdims=True)
    onehot = (iota == first).astype(jnp.float32)

    cand = _dotn(onehot, y2z)
    better = local_max > max_ref[...]
    max_ref[...] = jnp.where(better, local_max, max_ref[...])
    cand_ref[...] = jnp.where(better, cand, cand_ref[...])

    ones_y = jnp.ones((1, chunk), jnp.float32)
    ssq_ref[...] += _dotn(ones_y, y2z * y2z)


def _dn_step(x_ref, w_ref, age_ref, y2z_ref, out_ref,
             max_ref, cand_ref, ssq_ref, ypre_ref, y2zb_ref):
    i = pl.program_id(0)
    nsteps = pl.num_programs(0)

    @pl.when(i == 0)
    def _init():
        max_ref[...] = jnp.full_like(max_ref, -jnp.inf)
        cand_ref[...] = jnp.zeros_like(cand_ref)
        ssq_ref[...] = jnp.zeros_like(ssq_ref)
        ypre_ref[...] = jnp.full_like(ypre_ref, -jnp.inf)
        y2zb_ref[...] = jnp.zeros_like(y2zb_ref)

    _reduce_chunk(ypre_ref[...], y2zb_ref[...], max_ref, cand_ref, ssq_ref)

    xs = x_ref[...]
    w = w_ref[...]
    age = age_ref[...]
    b = xs.shape[0] // 2

    wh, wl = _split_bf16(w)
    p1 = _dotn(xs, wh)
    p2 = _dotn(xs, wl)
    dots = (p1[:b] + p1[b:]) + (p2[:b] + p2[b:])

    ones_x = jnp.ones((1, w.shape[1]), jnp.bfloat16)
    sqh, sql = _split_bf16(w * w)
    wssq = _dotn(ones_x, sqh) + _dotn(ones_x, sql)

    recip = 1.0 / jnp.maximum(jnp.sqrt(wssq), 1e-12)
    act = jnp.where(age >= 1.0, 1.0, 0.0)
    ypre_ref[...] = dots * (recip * act)
    y2zb_ref[...] = y2z_ref[...]

    @pl.when(i == nsteps - 1)
    def _fin():
        _reduce_chunk(ypre_ref[...], y2zb_ref[...], max_ref, cand_ref, ssq_ref)
        zn = jnp.maximum(jnp.sqrt(ssq_ref[...]), 1e-12)
        out_ref[...] = cand_ref[...] / zn


def kernel(x, z, per_item, x2y_w, z2y_w, y2z_w, y_neuron_age):
    batch = x.shape[0]
    xf = x.reshape(batch, -1)
    x_dim = xf.shape[1]
    y_num = x2y_w.shape[0]
    z_num = y2z_w.shape[0]
    grid = y_num // _Y_CHUNK

    xh = xf.astype(jnp.bfloat16)
    xl = (xf - xh.astype(jnp.float32)).astype(jnp.bfloat16)
    xs = jnp.concatenate([xh, xl], axis=0)

    return pl.pallas_call(
        _dn_step,
        grid=(grid,),
        in_specs=[
            pl.BlockSpec((2 * batch, x_dim), lambda i: (0, 0)),
            pl.BlockSpec((_Y_CHUNK, x_dim), lambda i: (i, 0)),
            pl.BlockSpec((1, _Y_CHUNK), lambda i: (0, i)),
            pl.BlockSpec((z_num, _Y_CHUNK), lambda i: (0, i)),
        ],
        out_specs=pl.BlockSpec((batch, z_num), lambda i: (0, 0)),
        out_shape=jax.ShapeDtypeStruct((batch, z_num), jnp.float32),
        scratch_shapes=[
            pltpu.VMEM((batch, 1), jnp.float32),
            pltpu.VMEM((batch, z_num), jnp.float32),
            pltpu.VMEM((1, z_num), jnp.float32),
            pltpu.VMEM((batch, _Y_CHUNK), jnp.float32),
            pltpu.VMEM((z_num, _Y_CHUNK), jnp.float32),
        ],
    )(xs, x2y_w, y_neuron_age, y2z_w)


# speedup vs baseline: 2.2438x; 1.3128x over previous
import jax
import jax.numpy as jnp
from jax.experimental import pallas as pl
from jax.experimental.pallas import tpu as pltpu

_Y_CHUNK = 4096


def _dotn(a, b):
    return jax.lax.dot_general(a, b, (((1,), (1,)), ((), ())),
                               preferred_element_type=jnp.float32)


def _dn_step(x_ref, w_ref, age_ref, y2z_ref, out_ref, max_ref, cand_ref, ssq_ref):
    i = pl.program_id(0)
    nsteps = pl.num_programs(0)

    @pl.when(i == 0)
    def _init():
        max_ref[...] = jnp.full_like(max_ref, -jnp.inf)
        cand_ref[...] = jnp.zeros_like(cand_ref)
        ssq_ref[...] = jnp.zeros_like(ssq_ref)

    xb = x_ref[...]
    w = w_ref[...]
    y2z = y2z_ref[...]
    age = age_ref[...]
    chunk = w.shape[0]

    rssq = jnp.sum(w * w, axis=1, keepdims=True)
    wn = w * (1.0 / jnp.maximum(jnp.sqrt(rssq), 1e-12))
    dots = _dotn(xb, wn.astype(jnp.bfloat16))

    act = jnp.where(age >= 1.0, 1.0, 0.0)
    y_pre = dots * act

    local_max = jnp.max(y_pre, axis=1, keepdims=True)
    iota = jax.lax.broadcasted_iota(jnp.int32, y_pre.shape, 1)
    eq = y_pre == local_max
    first = jnp.min(jnp.where(eq, iota, chunk), axis=1, keepdims=True)
    onehot = (iota == first).astype(jnp.float32)

    cand = _dotn(onehot, y2z)
    better = local_max > max_ref[...]
    max_ref[...] = jnp.where(better, local_max, max_ref[...])
    cand_ref[...] = jnp.where(better, cand, cand_ref[...])

    ones_y = jnp.ones((1, chunk), jnp.float32)
    ssq_ref[...] += _dotn(ones_y, y2z * y2z)

    @pl.when(i == nsteps - 1)
    def _fin():
        zn = jnp.maximum(jnp.sqrt(ssq_ref[...]), 1e-12)
        out_ref[...] = cand_ref[...] / zn


def kernel(x, z, per_item, x2y_w, z2y_w, y2z_w, y_neuron_age):
    batch = x.shape[0]
    xf = x.reshape(batch, -1)
    x_dim = xf.shape[1]
    y_num = x2y_w.shape[0]
    z_num = y2z_w.shape[0]
    grid = y_num // _Y_CHUNK

    xn = jnp.sqrt(jnp.sum(xf * xf, axis=1, keepdims=True))
    xfn = xf / jnp.maximum(xn, 1e-12)
    xb = xfn.astype(jnp.bfloat16)

    return pl.pallas_call(
        _dn_step,
        grid=(grid,),
        in_specs=[
            pl.BlockSpec((batch, x_dim), lambda i: (0, 0)),
            pl.BlockSpec((_Y_CHUNK, x_dim), lambda i: (i, 0)),
            pl.BlockSpec((1, _Y_CHUNK), lambda i: (0, i)),
            pl.BlockSpec((z_num, _Y_CHUNK), lambda i: (0, i)),
        ],
        out_specs=pl.BlockSpec((batch, z_num), lambda i: (0, 0)),
        out_shape=jax.ShapeDtypeStruct((batch, z_num), jnp.float32),
        scratch_shapes=[
            pltpu.VMEM((batch, 1), jnp.float32),
            pltpu.VMEM((batch, z_num), jnp.float32),
            pltpu.VMEM((1, z_num), jnp.float32),
        ],
    )(xb, x2y_w, y_neuron_age, y2z_w)
